# baseline (speedup 1.0000x reference)
import jax
import jax.numpy as jnp
from jax import lax
from jax.experimental import pallas as pl
from jax.experimental.pallas import tpu as pltpu
from jax.experimental.pallas import tpu_sc as plsc

UTT_LEN = 20
VOCAB1 = 100001
BATCH = 16384
EMB = 32
NUM_ROWS = UTT_LEN * VOCAB1
MAIN_ROWS = 2000000
TAIL = NUM_ROWS - MAIN_ROWS

NC = 2
NS = 16
L = 16
FPC = EMB // NC
BPT = BATCH // NS
EPT = UTT_LEN * BPT

CA = 1 << 20
CB = MAIN_ROWS - CA
SA = CA // NS
SB = CB // NS
ZSLOT = CA
BUF = CA + L

CH = 128
NSTREAM = EPT // CH
INFLIGHT = 8


def _body(utts_hbm, tablet_hbm, tail_hbm, out_hbm,
          buf_sh, idxa_v, idxb_v, idx19_v, g_v, tail_v, acc_v,
          dummy_v, zero_v, sem_stage, sem_gather, sem_out):
    cid = lax.axis_index("c")
    sid = lax.axis_index("s")
    b0 = sid * BPT
    base19 = (UTT_LEN - 1) * BPT

    for t in range(UTT_LEN):
        pltpu.sync_copy(utts_hbm.at[pl.ds(t * BATCH + b0, BPT)],
                        idxa_v.at[pl.ds(t * BPT, BPT)])
    pltpu.sync_copy(tail_hbm, tail_v)

    def _off(j, _):
        t = j // (BPT // L)
        sl = pl.ds(j * L, L)
        idxa_v[sl] = idxa_v[sl] + t * VOCAB1
        return 0

    lax.fori_loop(0, EPT // L, _off, 0)

    def _save19(j, _):
        idx19_v[pl.ds(j * L, L)] = idxa_v[pl.ds(base19 + j * L, L)]
        return 0

    lax.fori_loop(0, BPT // L, _save19, 0)

    def _split(j, _):
        sl = pl.ds(j * L, L)
        v = idxa_v[sl]
        in_a = v < CA
        in_b = jnp.logical_and(v >= CA, v < MAIN_ROWS)
        idxb_v[sl] = jnp.where(in_b, v - CA, ZSLOT)
        idxa_v[sl] = jnp.where(in_a, v, ZSLOT)
        return 0

    lax.fori_loop(0, EPT // L, _split, 0)

    @pl.when(sid == 0)
    def _zero_slot():
        zero_v[pl.ds(0, L)] = jnp.zeros((L,), jnp.float32)
        pltpu.sync_copy(zero_v, buf_sh.at[pl.ds(ZSLOT, L)])

    def _stage_a(fg):
        return pltpu.async_copy(tablet_hbm.at[fg, pl.ds(sid * SA, SA)],
                                buf_sh.at[pl.ds(sid * SA, SA)], sem_stage)

    def _stage_b(fg):
        return pltpu.async_copy(
            tablet_hbm.at[fg, pl.ds(CA + sid * SB, SB)],
            buf_sh.at[pl.ds(sid * SB, SB)], sem_stage)

    def _wait_a(fg):
        pltpu.make_async_copy(tablet_hbm.at[fg, pl.ds(sid * SA, SA)],
                              buf_sh.at[pl.ds(sid * SA, SA)],
                              sem_stage).wait()

    def _wait_b(fg):
        pltpu.make_async_copy(tablet_hbm.at[fg, pl.ds(CA + sid * SB, SB)],
                              buf_sh.at[pl.ds(sid * SB, SB)],
                              sem_stage).wait()

    def _drain_one():
        pltpu.make_async_copy(tablet_hbm.at[0, pl.ds(0, CH)],
                              dummy_v, sem_gather).wait()

    def _gather(idx_ref):
        def _one(k, _):
            pltpu.async_copy(buf_sh.at[idx_ref.at[pl.ds(k * CH, CH)]],
                             g_v.at[pl.ds(k * CH, CH)], sem_gather)

            @pl.when(k >= INFLIGHT)
            def _():
                _drain_one()

            return 0

        lax.fori_loop(0, NSTREAM, _one, 0)
        for _ in range(INFLIGHT):
            _drain_one()

    def _acc(first):
        def _one(jv, _):
            s = g_v[pl.ds(jv * L, L)]
            for t in range(1, UTT_LEN):
                s = s + g_v[pl.ds(t * BPT + jv * L, L)]
            sl = pl.ds(jv * L, L)
            if first:
                acc_v[sl] = s
            else:
                acc_v[sl] = acc_v[sl] + s
            return 0

        lax.fori_loop(0, BPT // L, _one, 0)

    def _patch_tail(fg):
        def _one(j, _):
            sl16 = pl.ds(j * L, L)
            iv = idx19_v[sl16]
            m = iv >= MAIN_ROWS
            tpos = jnp.clip(iv - MAIN_ROWS, 0, TAIL - 1) * EMB + fg
            tv = plsc.load_gather(tail_v, [tpos])
            cur = g_v[pl.ds(base19 + j * L, L)]
            g_v[pl.ds(base19 + j * L, L)] = jnp.where(m, tv, cur)
            return 0

        lax.fori_loop(0, BPT // L, _one, 0)

    _stage_a(cid * FPC)

    def _feature(f, _):
        fg = cid * FPC + f
        _wait_a(fg)
        plsc.subcore_barrier()
        _gather(idxa_v)
        plsc.subcore_barrier()
        _stage_b(fg)
        _acc(first=True)
        _wait_b(fg)
        plsc.subcore_barrier()
        _gather(idxb_v)
        plsc.subcore_barrier()

        @pl.when(f + 1 < FPC)
        def _():
            _stage_a(fg + 1)

        _patch_tail(fg)
        _acc(first=False)
        pltpu.async_copy(acc_v, out_hbm.at[pl.ds(fg * BATCH + b0, BPT)],
                         sem_out).wait()
        return 0

    lax.fori_loop(0, FPC, _feature, 0)


@jax.jit
def _emb_sum(utts1d, tablet, tail1d):
    fn = pl.kernel(
        _body,
        out_type=jax.ShapeDtypeStruct((EMB * BATCH,), jnp.float32),
        mesh=plsc.VectorSubcoreMesh(core_axis_name="c", subcore_axis_name="s",
                                    num_cores=NC, num_subcores=NS),
        scratch_types=[
            pltpu.VMEM_SHARED((BUF,), jnp.float32),
            pltpu.VMEM((EPT,), jnp.int32),
            pltpu.VMEM((EPT,), jnp.int32),
            pltpu.VMEM((BPT,), jnp.int32),
            pltpu.VMEM((EPT,), jnp.float32),
            pltpu.VMEM((TAIL * EMB,), jnp.float32),
            pltpu.VMEM((BPT,), jnp.float32),
            pltpu.VMEM((CH,), jnp.float32),
            pltpu.VMEM((L,), jnp.float32),
            pltpu.SemaphoreType.DMA,
            pltpu.SemaphoreType.DMA,
            pltpu.SemaphoreType.DMA,
        ],
        compiler_params=pltpu.CompilerParams(use_tc_tiling_on_sc=False,
                                             needs_layout_passes=False),
    )
    return fn(utts1d, tablet, tail1d)


def kernel(utts, table):
    utts1d = utts.astype(jnp.int32).reshape(-1)
    tablet = jnp.swapaxes(table, 0, 1)
    tail1d = table[MAIN_ROWS:].reshape(-1)
    out1d = _emb_sum(utts1d, tablet, tail1d)
    return out1d.reshape(EMB, BATCH).T.reshape(BATCH, EMB // 8, 8)


# speedup vs baseline: 1.0024x; 1.0024x over previous
import jax
import jax.numpy as jnp
from jax import lax
from jax.experimental import pallas as pl
from jax.experimental.pallas import tpu as pltpu
from jax.experimental.pallas import tpu_sc as plsc

UTT_LEN = 20
VOCAB1 = 100001
BATCH = 16384
EMB = 32
NUM_ROWS = UTT_LEN * VOCAB1
MAIN_ROWS = 2000000
TAIL = NUM_ROWS - MAIN_ROWS

NC = 2
NS = 16
L = 16
FPC = EMB // NC
BPT = BATCH // NS
EPT = UTT_LEN * BPT

CA = 1 << 20
CB = MAIN_ROWS - CA
SA = CA // NS
SB = CB // NS
ZSLOT = CA
BUF = CA + L

CH = 2048
NSTREAM = EPT // CH
INFLIGHT = 3


def _body(utts_hbm, tablet_hbm, tail_hbm, out_hbm,
          buf_sh, idxa_v, idxb_v, idx19_v, g_v, tail_v, acc_v,
          dummy_v, zero_v, sem_stage, sem_gather, sem_out):
    cid = lax.axis_index("c")
    sid = lax.axis_index("s")
    b0 = sid * BPT
    base19 = (UTT_LEN - 1) * BPT

    for t in range(UTT_LEN):
        pltpu.sync_copy(utts_hbm.at[pl.ds(t * BATCH + b0, BPT)],
                        idxa_v.at[pl.ds(t * BPT, BPT)])
    pltpu.sync_copy(tail_hbm, tail_v)

    def _off(j, _):
        t = j // (BPT // L)
        sl = pl.ds(j * L, L)
        idxa_v[sl] = idxa_v[sl] + t * VOCAB1
        return 0

    lax.fori_loop(0, EPT // L, _off, 0)

    def _save19(j, _):
        idx19_v[pl.ds(j * L, L)] = idxa_v[pl.ds(base19 + j * L, L)]
        return 0

    lax.fori_loop(0, BPT // L, _save19, 0)

    def _split(j, _):
        sl = pl.ds(j * L, L)
        v = idxa_v[sl]
        in_a = v < CA
        in_b = jnp.logical_and(v >= CA, v < MAIN_ROWS)
        idxb_v[sl] = jnp.where(in_b, v - CA, ZSLOT)
        idxa_v[sl] = jnp.where(in_a, v, ZSLOT)
        return 0

    lax.fori_loop(0, EPT // L, _split, 0)

    @pl.when(sid == 0)
    def _zero_slot():
        zero_v[pl.ds(0, L)] = jnp.zeros((L,), jnp.float32)
        pltpu.sync_copy(zero_v, buf_sh.at[pl.ds(ZSLOT, L)])

    def _stage_a(fg):
        return pltpu.async_copy(tablet_hbm.at[fg, pl.ds(sid * SA, SA)],
                                buf_sh.at[pl.ds(sid * SA, SA)], sem_stage)

    def _stage_b(fg):
        return pltpu.async_copy(
            tablet_hbm.at[fg, pl.ds(CA + sid * SB, SB)],
            buf_sh.at[pl.ds(sid * SB, SB)], sem_stage)

    def _wait_a(fg):
        pltpu.make_async_copy(tablet_hbm.at[fg, pl.ds(sid * SA, SA)],
                              buf_sh.at[pl.ds(sid * SA, SA)],
                              sem_stage).wait()

    def _wait_b(fg):
        pltpu.make_async_copy(tablet_hbm.at[fg, pl.ds(CA + sid * SB, SB)],
                              buf_sh.at[pl.ds(sid * SB, SB)],
                              sem_stage).wait()

    def _drain_one():
        pltpu.make_async_copy(tablet_hbm.at[0, pl.ds(0, CH)],
                              dummy_v, sem_gather).wait()

    def _gather(idx_ref):
        def _one(k, _):
            pltpu.async_copy(buf_sh.at[idx_ref.at[pl.ds(k * CH, CH)]],
                             g_v.at[pl.ds(k * CH, CH)], sem_gather)

            @pl.when(k >= INFLIGHT)
            def _():
                _drain_one()

            return 0

        lax.fori_loop(0, NSTREAM, _one, 0)
        for _ in range(INFLIGHT):
            _drain_one()

    def _acc(first):
        def _one(jv, _):
            s = g_v[pl.ds(jv * L, L)]
            for t in range(1, UTT_LEN):
                s = s + g_v[pl.ds(t * BPT + jv * L, L)]
            sl = pl.ds(jv * L, L)
            if first:
                acc_v[sl] = s
            else:
                acc_v[sl] = acc_v[sl] + s
            return 0

        lax.fori_loop(0, BPT // L, _one, 0)

    def _patch_tail(fg):
        def _one(j, _):
            sl16 = pl.ds(j * L, L)
            iv = idx19_v[sl16]
            m = iv >= MAIN_ROWS
            tpos = jnp.clip(iv - MAIN_ROWS, 0, TAIL - 1) + fg * TAIL
            tv = plsc.load_gather(tail_v, [tpos])
            cur = g_v[pl.ds(base19 + j * L, L)]
            g_v[pl.ds(base19 + j * L, L)] = jnp.where(m, tv, cur)
            return 0

        lax.fori_loop(0, BPT // L, _one, 0)

    _stage_a(cid * FPC)

    def _feature(f, _):
        fg = cid * FPC + f
        _wait_a(fg)
        plsc.subcore_barrier()
        _gather(idxa_v)
        plsc.subcore_barrier()
        _stage_b(fg)
        _acc(first=True)
        _wait_b(fg)
        plsc.subcore_barrier()
        _gather(idxb_v)
        plsc.subcore_barrier()

        @pl.when(f + 1 < FPC)
        def _():
            _stage_a(fg + 1)

        _patch_tail(fg)
        _acc(first=False)
        pltpu.async_copy(acc_v, out_hbm.at[pl.ds(fg * BATCH + b0, BPT)],
                         sem_out).wait()
        return 0

    lax.fori_loop(0, FPC, _feature, 0)


@jax.jit
def _emb_sum(utts1d, tablet, tail1d):
    fn = pl.kernel(
        _body,
        out_type=jax.ShapeDtypeStruct((EMB * BATCH,), jnp.float32),
        mesh=plsc.VectorSubcoreMesh(core_axis_name="c", subcore_axis_name="s",
                                    num_cores=NC, num_subcores=NS),
        scratch_types=[
            pltpu.VMEM_SHARED((BUF,), jnp.float32),
            pltpu.VMEM((EPT,), jnp.int32),
            pltpu.VMEM((EPT,), jnp.int32),
            pltpu.VMEM((BPT,), jnp.int32),
            pltpu.VMEM((EPT,), jnp.float32),
            pltpu.VMEM((TAIL * EMB,), jnp.float32),
            pltpu.VMEM((BPT,), jnp.float32),
            pltpu.VMEM((CH,), jnp.float32),
            pltpu.VMEM((L,), jnp.float32),
            pltpu.SemaphoreType.DMA,
            pltpu.SemaphoreType.DMA,
            pltpu.SemaphoreType.DMA,
        ],
        compiler_params=pltpu.CompilerParams(use_tc_tiling_on_sc=False,
                                             needs_layout_passes=False),
    )
    return fn(utts1d, tablet, tail1d)


def kernel(utts, table):
    utts1d = utts.astype(jnp.int32).reshape(-1)
    tablet = jnp.swapaxes(table, 0, 1)
    tail1d = lax.slice(tablet, (0, MAIN_ROWS), (EMB, NUM_ROWS)).reshape(-1)
    out1d = _emb_sum(utts1d, tablet, tail1d)
    return out1d.reshape(EMB, BATCH).T.reshape(BATCH, EMB // 8, 8)


# speedup vs baseline: 6.0464x; 6.0321x over previous
import jax
import jax.numpy as jnp
from jax import lax
from jax.experimental import pallas as pl
from jax.experimental.pallas import tpu as pltpu
from jax.experimental.pallas import tpu_sc as plsc

UTT_LEN = 20
VOCAB1 = 100001
BATCH = 16384
EMB = 32
NUM_ROWS = UTT_LEN * VOCAB1
PACK = 4
PROWS = NUM_ROWS // PACK

NC = 2
NS = 16
L = 16
NW = NC * NS
BPW = BATCH // NW
EPW = UTT_LEN * BPW
CH = 128
NPH = EPW // CH
PEMB = PACK * EMB


def _body(utts_hbm, table_hbm, out_hbm, idx_v, idx4_v, rows_v, acc_v,
          sem0, sem1):
    cid = lax.axis_index("c")
    sid = lax.axis_index("s")
    wid = sid * NC + cid
    base = wid * BPW

    for t in range(UTT_LEN):
        pltpu.sync_copy(utts_hbm.at[pl.ds(t * BATCH + base, BPW)],
                        idx_v.at[pl.ds(t * BPW, BPW)])

    def _prep(j, _):
        sl = pl.ds(j * L, L)
        v = idx_v[sl] + (j // (BPW // L)) * VOCAB1
        idx_v[sl] = v
        idx4_v[sl] = v >> 2
        return 0

    lax.fori_loop(0, EPW // L, _prep, 0)

    def _zero(j, _):
        acc_v[pl.ds(j * L, L)] = jnp.zeros((L,), jnp.float32)
        return 0

    lax.fori_loop(0, BPW * EMB // L, _zero, 0)

    sems = (sem0, sem1)

    def fire(p, par):
        pltpu.async_copy(table_hbm.at[idx4_v.at[pl.ds(p * CH, CH)]],
                         rows_v.at[par], sems[par])

    def wait(par):
        pltpu.make_async_copy(table_hbm.at[pl.ds(0, CH)],
                              rows_v.at[par], sems[par]).wait()

    lanes = lax.iota(jnp.int32, L)

    def accum(p, par):
        rv = rows_v.at[par]
        bb = lax.rem(p * CH, BPW)

        def _grp(g, _):
            e0 = p * CH + g * L
            iv = idx_v[pl.ds(e0, L)]
            rows = g * L + lanes
            cols = (iv & (PACK - 1)) * EMB
            dstb = (bb + g * L + lanes) * EMB
            for cc in range(EMB):
                val = plsc.load_gather(rv, [rows, cols + cc])
                plsc.addupdate_scatter(acc_v, [dstb + cc], val)
            return 0

        lax.fori_loop(0, CH // L, _grp, 0)

    fire(0, 0)
    fire(1, 1)

    def _phase2(i, _):
        p = i * 2
        wait(0)
        accum(p, 0)

        @pl.when(p + 2 < NPH)
        def _():
            fire(p + 2, 0)

        wait(1)
        accum(p + 1, 1)

        @pl.when(p + 3 < NPH)
        def _():
            fire(p + 3, 1)

        return 0

    lax.fori_loop(0, NPH // 2, _phase2, 0)

    pltpu.sync_copy(acc_v, out_hbm.at[pl.ds(base * EMB, BPW * EMB)])


@jax.jit
def _emb_sum(utts1d, table128):
    fn = pl.kernel(
        _body,
        out_type=jax.ShapeDtypeStruct((BATCH * EMB,), jnp.float32),
        mesh=plsc.VectorSubcoreMesh(core_axis_name="c", subcore_axis_name="s",
                                    num_cores=NC, num_subcores=NS),
        scratch_types=[
            pltpu.VMEM((EPW,), jnp.int32),
            pltpu.VMEM((EPW,), jnp.int32),
            pltpu.VMEM((2, CH, PEMB), jnp.float32),
            pltpu.VMEM((BPW * EMB,), jnp.float32),
            pltpu.SemaphoreType.DMA,
            pltpu.SemaphoreType.DMA,
        ],
        compiler_params=pltpu.CompilerParams(needs_layout_passes=False),
    )
    return fn(utts1d, table128)


def kernel(utts, table):
    utts1d = utts.astype(jnp.int32).reshape(-1)
    table128 = table.reshape(PROWS, PEMB)
    out1d = _emb_sum(utts1d, table128)
    return out1d.reshape(BATCH, EMB // 8, 8)
